# Initial kernel scaffold; baseline (speedup 1.0000x reference)
import jax
import jax.numpy as jnp
from jax.experimental import pallas as pl


def kernel(x, adj_t, W, b):
    raise NotImplementedError("write your pallas kernel here")


# speedup vs baseline: 20.1475x; 20.1475x over previous
import functools
import jax
import jax.numpy as jnp
from jax import lax
from jax.experimental import pallas as pl
from jax.experimental.pallas import tpu as pltpu
from jax.experimental.pallas import tpu_sc as plsc

N = 10000
E = 320000
D = 128
C = 40

NP = 10240
NC, NS = 2, 16
NW = NC * NS
CHUNK = 128
NCHUNK = 80
EP = NW * NCHUNK * CHUNK
ROWS_PER_TILE = NP // NS


def _mesh():
    return plsc.VectorSubcoreMesh(core_axis_name="c", subcore_axis_name="s")


@functools.partial(
    pl.kernel,
    out_type=jax.ShapeDtypeStruct((NC, NP), jnp.float32),
    mesh=_mesh(),
    scratch_types=[
        pltpu.VMEM((NCHUNK, CHUNK), jnp.int32),
        pltpu.VMEM((CHUNK,), jnp.float32),
        pltpu.VMEM((ROWS_PER_TILE,), jnp.float32),
        pltpu.VMEM_SHARED((NP,), jnp.float32),
    ],
    compiler_params=pltpu.CompilerParams(use_tc_tiling_on_sc=False),
)
def _deg_kernel(cidx_hbm, out_hbm, cidx_v, ones_v, vbuf, acc_sh):
    cid = lax.axis_index("c")
    sid = lax.axis_index("s")
    wid = sid * NC + cid
    base = sid * ROWS_PER_TILE

    def fill(i, _):
        ones_v[pl.ds(i * 16, 16)] = jnp.ones((16,), jnp.float32)
        vbuf[pl.ds(i * 16, 16)] = jnp.zeros((16,), jnp.float32)
        return 0
    lax.fori_loop(0, CHUNK // 16, fill, 0)

    def zero(i, _):
        vbuf[pl.ds(i * 16, 16)] = jnp.zeros((16,), jnp.float32)
        return 0
    lax.fori_loop(0, ROWS_PER_TILE // 16, zero, 0)

    pltpu.sync_copy(vbuf, acc_sh.at[pl.ds(base, ROWS_PER_TILE)])
    pltpu.sync_copy(cidx_hbm.at[wid], cidx_v)
    plsc.subcore_barrier()

    def body(i, _):
        pltpu.sync_copy(ones_v, acc_sh.at[cidx_v.at[i]], add=True)
        return 0
    lax.fori_loop(0, NCHUNK, body, 0)

    plsc.subcore_barrier()
    pltpu.sync_copy(acc_sh.at[pl.ds(base, ROWS_PER_TILE)], vbuf)
    pltpu.sync_copy(vbuf, out_hbm.at[cid, pl.ds(base, ROWS_PER_TILE)])


@functools.partial(
    pl.kernel,
    out_type=jax.ShapeDtypeStruct((NC, NP, C), jnp.float32),
    mesh=_mesh(),
    scratch_types=[
        pltpu.VMEM((NCHUNK, CHUNK), jnp.int32),
        pltpu.VMEM((NCHUNK, CHUNK), jnp.int32),
        pltpu.VMEM((CHUNK, C), jnp.float32),
        pltpu.VMEM((ROWS_PER_TILE, C), jnp.float32),
        pltpu.VMEM_SHARED((NP, C), jnp.float32),
    ],
    compiler_params=pltpu.CompilerParams(use_tc_tiling_on_sc=False),
)
def _hop_kernel(u_hbm, ridx_hbm, cidx_hbm, out_hbm, ridx_v, cidx_v, rows_v,
                vbuf, acc_sh):
    cid = lax.axis_index("c")
    sid = lax.axis_index("s")
    wid = sid * NC + cid
    base = sid * ROWS_PER_TILE

    pltpu.sync_copy(u_hbm.at[pl.ds(base, ROWS_PER_TILE)], vbuf)
    pltpu.sync_copy(vbuf, acc_sh.at[pl.ds(base, ROWS_PER_TILE)])
    pltpu.sync_copy(ridx_hbm.at[wid], ridx_v)
    pltpu.sync_copy(cidx_hbm.at[wid], cidx_v)
    plsc.subcore_barrier()

    def body(i, _):
        pltpu.sync_copy(u_hbm.at[ridx_v.at[i]], rows_v)
        pltpu.sync_copy(rows_v, acc_sh.at[cidx_v.at[i]], add=True)
        return 0
    lax.fori_loop(0, NCHUNK, body, 0)

    plsc.subcore_barrier()
    pltpu.sync_copy(acc_sh.at[pl.ds(base, ROWS_PER_TILE)], vbuf)
    pltpu.sync_copy(vbuf, out_hbm.at[cid, pl.ds(base, ROWS_PER_TILE)])


def _proj_body(x_ref, w_ref, degp_ref, u0_ref, dinv_ref):
    deg = degp_ref[0, :] + degp_ref[1, :] + 1.0
    dv = lax.rsqrt(deg)
    z = lax.dot_general(x_ref[...], w_ref[...], (((1,), (1,)), ((), ())),
                        preferred_element_type=jnp.float32)
    u0_ref[...] = z * dv[:, None]
    dinv_ref[...] = dv


def _proj(x_p, W, degp):
    blk = 1024
    return pl.pallas_call(
        _proj_body,
        grid=(NP // blk,),
        in_specs=[
            pl.BlockSpec((blk, D), lambda i: (i, 0)),
            pl.BlockSpec((C, D), lambda i: (0, 0)),
            pl.BlockSpec((NC, blk), lambda i: (0, i)),
        ],
        out_specs=[
            pl.BlockSpec((blk, C), lambda i: (i, 0)),
            pl.BlockSpec((blk,), lambda i: (i,)),
        ],
        out_shape=[
            jax.ShapeDtypeStruct((NP, C), jnp.float32),
            jax.ShapeDtypeStruct((NP,), jnp.float32),
        ],
    )(x_p, W, degp)


def _mid_body(p_ref, u_ref, dinv_ref, out_ref):
    s = p_ref[0] + p_ref[1] - u_ref[...]
    dv = dinv_ref[...]
    out_ref[...] = s * (dv * dv)[:, None]


def _mid(parts, u, dinv):
    blk = 1024
    return pl.pallas_call(
        _mid_body,
        grid=(NP // blk,),
        in_specs=[
            pl.BlockSpec((NC, blk, C), lambda i: (0, i, 0)),
            pl.BlockSpec((blk, C), lambda i: (i, 0)),
            pl.BlockSpec((blk,), lambda i: (i,)),
        ],
        out_specs=pl.BlockSpec((blk, C), lambda i: (i, 0)),
        out_shape=jax.ShapeDtypeStruct((NP, C), jnp.float32),
    )(parts, u, dinv)


def _fin_body(p_ref, u_ref, dinv_ref, b_ref, out_ref):
    s = p_ref[0] + p_ref[1] - u_ref[...]
    out_ref[...] = s * dinv_ref[...][:, None] + b_ref[...][None, :]


def _fin(parts, u, dinv, b):
    blk = 1024
    return pl.pallas_call(
        _fin_body,
        grid=(NP // blk,),
        in_specs=[
            pl.BlockSpec((NC, blk, C), lambda i: (0, i, 0)),
            pl.BlockSpec((blk, C), lambda i: (i, 0)),
            pl.BlockSpec((blk,), lambda i: (i,)),
            pl.BlockSpec((C,), lambda i: (0,)),
        ],
        out_specs=pl.BlockSpec((blk, C), lambda i: (i, 0)),
        out_shape=jax.ShapeDtypeStruct((NP, C), jnp.float32),
    )(parts, u, dinv, b)


@jax.jit
def kernel(x, adj_t, W, b):
    row = adj_t[0].astype(jnp.int32)
    col = adj_t[1].astype(jnp.int32)
    pad = jnp.full((EP - E,), NP - 1, jnp.int32)
    row_p = jnp.concatenate([row, pad]).reshape(NW, NCHUNK, CHUNK)
    col_p = jnp.concatenate([col, pad]).reshape(NW, NCHUNK, CHUNK)
    x_p = jnp.pad(x, ((0, NP - N), (0, 0)))

    degp = _deg_kernel(col_p)
    u0, dinv = _proj(x_p, W, degp)
    p1 = _hop_kernel(u0, row_p, col_p)
    t2 = _mid(p1, u0, dinv)
    p2 = _hop_kernel(t2, row_p, col_p)
    out = _fin(p2, t2, dinv, b)
    return out[:N]


# speedup vs baseline: 23.1369x; 1.1484x over previous
import functools
import jax
import jax.numpy as jnp
from jax import lax
from jax.experimental import pallas as pl
from jax.experimental.pallas import tpu as pltpu
from jax.experimental.pallas import tpu_sc as plsc

N = 10000
E = 320000
D = 128
C = 40

NP = 10240
NC, NS = 2, 16
NW = NC * NS
CHUNK = 128
NCHUNK = 80
EP = NW * NCHUNK * CHUNK
ROWS_PER_TILE = NP // NS


def _mesh():
    return plsc.VectorSubcoreMesh(core_axis_name="c", subcore_axis_name="s")


@functools.partial(
    pl.kernel,
    out_type=jax.ShapeDtypeStruct((NC, NP), jnp.float32),
    mesh=_mesh(),
    scratch_types=[
        pltpu.VMEM((NCHUNK, CHUNK), jnp.int32),
        pltpu.VMEM((CHUNK,), jnp.float32),
        pltpu.VMEM((ROWS_PER_TILE,), jnp.float32),
        pltpu.VMEM_SHARED((NP,), jnp.float32),
    ],
    compiler_params=pltpu.CompilerParams(use_tc_tiling_on_sc=False),
)
def _deg_kernel(cidx_hbm, out_hbm, cidx_v, ones_v, vbuf, acc_sh):
    cid = lax.axis_index("c")
    sid = lax.axis_index("s")
    wid = sid * NC + cid
    base = sid * ROWS_PER_TILE

    def fill(i, _):
        ones_v[pl.ds(i * 16, 16)] = jnp.ones((16,), jnp.float32)
        vbuf[pl.ds(i * 16, 16)] = jnp.zeros((16,), jnp.float32)
        return 0
    lax.fori_loop(0, CHUNK // 16, fill, 0)

    def zero(i, _):
        vbuf[pl.ds(i * 16, 16)] = jnp.zeros((16,), jnp.float32)
        return 0
    lax.fori_loop(0, ROWS_PER_TILE // 16, zero, 0)

    pltpu.sync_copy(vbuf, acc_sh.at[pl.ds(base, ROWS_PER_TILE)])
    pltpu.sync_copy(cidx_hbm.at[wid], cidx_v)
    plsc.subcore_barrier()

    def body(i, _):
        pltpu.sync_copy(ones_v, acc_sh.at[cidx_v.at[i]], add=True)
        return 0
    lax.fori_loop(0, NCHUNK, body, 0)

    plsc.subcore_barrier()
    pltpu.sync_copy(acc_sh.at[pl.ds(base, ROWS_PER_TILE)], vbuf)
    pltpu.sync_copy(vbuf, out_hbm.at[cid, pl.ds(base, ROWS_PER_TILE)])


NBUF = 4

@functools.partial(
    pl.kernel,
    out_type=jax.ShapeDtypeStruct((NC, NP, C), jnp.float32),
    mesh=_mesh(),
    scratch_types=[
        pltpu.VMEM((NCHUNK, CHUNK), jnp.int32),
        pltpu.VMEM((NCHUNK, CHUNK), jnp.int32),
        [pltpu.VMEM((CHUNK, C), jnp.float32) for _ in range(NBUF)],
        [pltpu.SemaphoreType.DMA for _ in range(NBUF)],
        [pltpu.SemaphoreType.DMA for _ in range(NBUF)],
        pltpu.VMEM((ROWS_PER_TILE, C), jnp.float32),
        pltpu.VMEM_SHARED((NP, C), jnp.float32),
    ],
    compiler_params=pltpu.CompilerParams(use_tc_tiling_on_sc=False),
)
def _hop_kernel(u_hbm, ridx_hbm, cidx_hbm, out_hbm, ridx_v, cidx_v, bufs,
                gsems, ssems, vbuf, acc_sh):
    cid = lax.axis_index("c")
    sid = lax.axis_index("s")
    wid = sid * NC + cid
    base = sid * ROWS_PER_TILE

    pltpu.sync_copy(ridx_hbm.at[wid], ridx_v)
    pltpu.sync_copy(cidx_hbm.at[wid], cidx_v)
    for b in range(NBUF):
        pltpu.async_copy(u_hbm.at[ridx_v.at[b]], bufs[b], gsems[b])

    pltpu.sync_copy(u_hbm.at[pl.ds(base, ROWS_PER_TILE)], vbuf)
    pltpu.sync_copy(vbuf, acc_sh.at[pl.ds(base, ROWS_PER_TILE)])
    plsc.subcore_barrier()

    def body(i, _):
        for b in range(NBUF):
            c = i * NBUF + b
            pltpu.make_async_copy(u_hbm.at[ridx_v.at[c]], bufs[b],
                                  gsems[b]).wait()
            pltpu.async_copy(bufs[b], acc_sh.at[cidx_v.at[c]], ssems[b],
                             add=True)
        for b in range(NBUF):
            c = i * NBUF + b
            cn = jnp.minimum(c + NBUF, NCHUNK - 1)
            pltpu.make_async_copy(bufs[b], acc_sh.at[cidx_v.at[c]],
                                  ssems[b]).wait()
            pltpu.async_copy(u_hbm.at[ridx_v.at[cn]], bufs[b], gsems[b])
        return 0
    lax.fori_loop(0, NCHUNK // NBUF, body, 0)

    for b in range(NBUF):
        pltpu.make_async_copy(u_hbm.at[ridx_v.at[NCHUNK - 1]], bufs[b],
                              gsems[b]).wait()

    plsc.subcore_barrier()
    pltpu.sync_copy(acc_sh.at[pl.ds(base, ROWS_PER_TILE)], vbuf)
    pltpu.sync_copy(vbuf, out_hbm.at[cid, pl.ds(base, ROWS_PER_TILE)])


def _proj_body(x_ref, w_ref, degp_ref, u0_ref, dinv_ref):
    deg = degp_ref[0, :] + degp_ref[1, :] + 1.0
    dv = lax.rsqrt(deg)
    z = lax.dot_general(x_ref[...], w_ref[...], (((1,), (1,)), ((), ())),
                        preferred_element_type=jnp.float32)
    u0_ref[...] = z * dv[:, None]
    dinv_ref[...] = dv


def _proj(x_p, W, degp):
    blk = 1024
    return pl.pallas_call(
        _proj_body,
        grid=(NP // blk,),
        in_specs=[
            pl.BlockSpec((blk, D), lambda i: (i, 0)),
            pl.BlockSpec((C, D), lambda i: (0, 0)),
            pl.BlockSpec((NC, blk), lambda i: (0, i)),
        ],
        out_specs=[
            pl.BlockSpec((blk, C), lambda i: (i, 0)),
            pl.BlockSpec((blk,), lambda i: (i,)),
        ],
        out_shape=[
            jax.ShapeDtypeStruct((NP, C), jnp.float32),
            jax.ShapeDtypeStruct((NP,), jnp.float32),
        ],
    )(x_p, W, degp)


def _mid_body(p_ref, u_ref, dinv_ref, out_ref):
    s = p_ref[0] + p_ref[1] - u_ref[...]
    dv = dinv_ref[...]
    out_ref[...] = s * (dv * dv)[:, None]


def _mid(parts, u, dinv):
    blk = 1024
    return pl.pallas_call(
        _mid_body,
        grid=(NP // blk,),
        in_specs=[
            pl.BlockSpec((NC, blk, C), lambda i: (0, i, 0)),
            pl.BlockSpec((blk, C), lambda i: (i, 0)),
            pl.BlockSpec((blk,), lambda i: (i,)),
        ],
        out_specs=pl.BlockSpec((blk, C), lambda i: (i, 0)),
        out_shape=jax.ShapeDtypeStruct((NP, C), jnp.float32),
    )(parts, u, dinv)


def _fin_body(p_ref, u_ref, dinv_ref, b_ref, out_ref):
    s = p_ref[0] + p_ref[1] - u_ref[...]
    out_ref[...] = s * dinv_ref[...][:, None] + b_ref[...][None, :]


def _fin(parts, u, dinv, b):
    blk = 1024
    return pl.pallas_call(
        _fin_body,
        grid=(NP // blk,),
        in_specs=[
            pl.BlockSpec((NC, blk, C), lambda i: (0, i, 0)),
            pl.BlockSpec((blk, C), lambda i: (i, 0)),
            pl.BlockSpec((blk,), lambda i: (i,)),
            pl.BlockSpec((C,), lambda i: (0,)),
        ],
        out_specs=pl.BlockSpec((blk, C), lambda i: (i, 0)),
        out_shape=jax.ShapeDtypeStruct((NP, C), jnp.float32),
    )(parts, u, dinv, b)


@jax.jit
def kernel(x, adj_t, W, b):
    row = adj_t[0].astype(jnp.int32)
    col = adj_t[1].astype(jnp.int32)
    pad = jnp.full((EP - E,), NP - 1, jnp.int32)
    row_p = jnp.concatenate([row, pad]).reshape(NW, NCHUNK, CHUNK)
    col_p = jnp.concatenate([col, pad]).reshape(NW, NCHUNK, CHUNK)
    x_p = jnp.pad(x, ((0, NP - N), (0, 0)))

    degp = _deg_kernel(col_p)
    u0, dinv = _proj(x_p, W, degp)
    p1 = _hop_kernel(u0, row_p, col_p)
    t2 = _mid(p1, u0, dinv)
    p2 = _hop_kernel(t2, row_p, col_p)
    out = _fin(p2, t2, dinv, b)
    return out[:N]


# speedup vs baseline: 23.2176x; 1.0035x over previous
import functools
import jax
import jax.numpy as jnp
from jax import lax
from jax.experimental import pallas as pl
from jax.experimental.pallas import tpu as pltpu
from jax.experimental.pallas import tpu_sc as plsc

N = 10000
E = 320000
D = 128
C = 40

NP = 10240
NC, NS = 2, 16
NW = NC * NS
CHUNK = 128
TCHUNKS = 2560
EP = TCHUNKS * CHUNK
NC0 = 120
NC1 = 40
ROWS_PER_TILE = NP // NS
NBUF = 4


def _mesh():
    return plsc.VectorSubcoreMesh(core_axis_name="c", subcore_axis_name="s")


@functools.partial(
    pl.kernel,
    out_type=jax.ShapeDtypeStruct((NC, NP), jnp.float32),
    mesh=_mesh(),
    scratch_types=[
        pltpu.VMEM((TCHUNKS // NW, CHUNK), jnp.int32),
        pltpu.VMEM((CHUNK,), jnp.float32),
        pltpu.VMEM((ROWS_PER_TILE,), jnp.float32),
        pltpu.VMEM_SHARED((NP,), jnp.float32),
    ],
    compiler_params=pltpu.CompilerParams(use_tc_tiling_on_sc=False),
)
def _deg_kernel(cidx_hbm, out_hbm, cidx_v, ones_v, vbuf, acc_sh):
    cid = lax.axis_index("c")
    sid = lax.axis_index("s")
    wid = sid * NC + cid
    base = sid * ROWS_PER_TILE
    nck = TCHUNKS // NW

    def fill(i, _):
        ones_v[pl.ds(i * 16, 16)] = jnp.ones((16,), jnp.float32)
        return 0
    lax.fori_loop(0, CHUNK // 16, fill, 0)

    def zero(i, _):
        vbuf[pl.ds(i * 16, 16)] = jnp.zeros((16,), jnp.float32)
        return 0
    lax.fori_loop(0, ROWS_PER_TILE // 16, zero, 0)

    pltpu.sync_copy(vbuf, acc_sh.at[pl.ds(base, ROWS_PER_TILE)])
    pltpu.sync_copy(cidx_hbm.at[pl.ds(wid * nck, nck)], cidx_v)
    plsc.subcore_barrier()

    def body(i, _):
        pltpu.sync_copy(ones_v, acc_sh.at[cidx_v.at[i]], add=True)
        return 0
    lax.fori_loop(0, nck, body, 0)

    plsc.subcore_barrier()
    pltpu.sync_copy(acc_sh.at[pl.ds(base, ROWS_PER_TILE)], vbuf)
    pltpu.sync_copy(vbuf, out_hbm.at[cid, pl.ds(base, ROWS_PER_TILE)])


@functools.partial(
    pl.kernel,
    out_type=jax.ShapeDtypeStruct((NC, NP, C), jnp.float32),
    mesh=_mesh(),
    scratch_types=[
        pltpu.VMEM((NC0, CHUNK), jnp.int32),
        pltpu.VMEM((NC0, CHUNK), jnp.int32),
        [pltpu.VMEM((CHUNK, C), jnp.float32) for _ in range(NBUF)],
        [pltpu.SemaphoreType.DMA for _ in range(NBUF)],
        [pltpu.SemaphoreType.DMA for _ in range(NBUF)],
        pltpu.VMEM((ROWS_PER_TILE, C), jnp.float32),
        pltpu.VMEM_SHARED((NP, C), jnp.float32),
    ],
    compiler_params=pltpu.CompilerParams(use_tc_tiling_on_sc=False),
)
def _hop_kernel(u_hbm, ridx_hbm, cidx_hbm, out_hbm, ridx_v, cidx_v, bufs,
                gsems, ssems, vbuf, acc_sh):
    cid = lax.axis_index("c")
    sid = lax.axis_index("s")
    base = sid * ROWS_PER_TILE

    def pipeline(nchunks, off):
        pltpu.sync_copy(ridx_hbm.at[pl.ds(off, nchunks)],
                        ridx_v.at[pl.ds(0, nchunks)])
        pltpu.sync_copy(cidx_hbm.at[pl.ds(off, nchunks)],
                        cidx_v.at[pl.ds(0, nchunks)])
        for b in range(NBUF):
            pltpu.async_copy(u_hbm.at[ridx_v.at[b]], bufs[b], gsems[b])

        pltpu.sync_copy(u_hbm.at[pl.ds(base, ROWS_PER_TILE)], vbuf)
        pltpu.sync_copy(vbuf, acc_sh.at[pl.ds(base, ROWS_PER_TILE)])
        plsc.subcore_barrier()

        def body(i, _):
            for b in range(NBUF):
                c = i * NBUF + b
                pltpu.make_async_copy(u_hbm.at[ridx_v.at[c]], bufs[b],
                                      gsems[b]).wait()
                pltpu.async_copy(bufs[b], acc_sh.at[cidx_v.at[c]], ssems[b],
                                 add=True)
            for b in range(NBUF):
                c = i * NBUF + b
                cn = jnp.minimum(c + NBUF, nchunks - 1)
                pltpu.make_async_copy(bufs[b], acc_sh.at[cidx_v.at[c]],
                                      ssems[b]).wait()
                pltpu.async_copy(u_hbm.at[ridx_v.at[cn]], bufs[b], gsems[b])
            return 0
        lax.fori_loop(0, nchunks // NBUF, body, 0)

        for b in range(NBUF):
            pltpu.make_async_copy(u_hbm.at[ridx_v.at[nchunks - 1]], bufs[b],
                                  gsems[b]).wait()
        plsc.subcore_barrier()
        pltpu.sync_copy(acc_sh.at[pl.ds(base, ROWS_PER_TILE)], vbuf)
        pltpu.sync_copy(vbuf, out_hbm.at[cid, pl.ds(base, ROWS_PER_TILE)])

    @pl.when(cid == 0)
    def _():
        pipeline(NC0, sid * NC0)

    @pl.when(cid == 1)
    def _():
        pipeline(NC1, NS * NC0 + sid * NC1)


def _proj_body(x_ref, w_ref, degp_ref, u0_ref, dinv_ref):
    deg = degp_ref[0, :] + degp_ref[1, :] + 1.0
    dv = lax.rsqrt(deg)
    z = lax.dot_general(x_ref[...], w_ref[...], (((1,), (1,)), ((), ())),
                        preferred_element_type=jnp.float32)
    u0_ref[...] = z * dv[:, None]
    dinv_ref[...] = dv


def _proj(x_p, W, degp):
    blk = 1024
    return pl.pallas_call(
        _proj_body,
        grid=(NP // blk,),
        in_specs=[
            pl.BlockSpec((blk, D), lambda i: (i, 0)),
            pl.BlockSpec((C, D), lambda i: (0, 0)),
            pl.BlockSpec((NC, blk), lambda i: (0, i)),
        ],
        out_specs=[
            pl.BlockSpec((blk, C), lambda i: (i, 0)),
            pl.BlockSpec((blk,), lambda i: (i,)),
        ],
        out_shape=[
            jax.ShapeDtypeStruct((NP, C), jnp.float32),
            jax.ShapeDtypeStruct((NP,), jnp.float32),
        ],
    )(x_p, W, degp)


def _mid_body(p_ref, u_ref, dinv_ref, out_ref):
    s = p_ref[0] + p_ref[1] - u_ref[...]
    dv = dinv_ref[...]
    out_ref[...] = s * (dv * dv)[:, None]


def _mid(parts, u, dinv):
    blk = 1024
    return pl.pallas_call(
        _mid_body,
        grid=(NP // blk,),
        in_specs=[
            pl.BlockSpec((NC, blk, C), lambda i: (0, i, 0)),
            pl.BlockSpec((blk, C), lambda i: (i, 0)),
            pl.BlockSpec((blk,), lambda i: (i,)),
        ],
        out_specs=pl.BlockSpec((blk, C), lambda i: (i, 0)),
        out_shape=jax.ShapeDtypeStruct((NP, C), jnp.float32),
    )(parts, u, dinv)


def _fin_body(p_ref, u_ref, dinv_ref, b_ref, out_ref):
    s = p_ref[0] + p_ref[1] - u_ref[...]
    out_ref[...] = s * dinv_ref[...][:, None] + b_ref[...][None, :]


def _fin(parts, u, dinv, b):
    blk = 1024
    return pl.pallas_call(
        _fin_body,
        grid=(NP // blk,),
        in_specs=[
            pl.BlockSpec((NC, blk, C), lambda i: (0, i, 0)),
            pl.BlockSpec((blk, C), lambda i: (i, 0)),
            pl.BlockSpec((blk,), lambda i: (i,)),
            pl.BlockSpec((C,), lambda i: (0,)),
        ],
        out_specs=pl.BlockSpec((blk, C), lambda i: (i, 0)),
        out_shape=jax.ShapeDtypeStruct((NP, C), jnp.float32),
    )(parts, u, dinv, b)


@jax.jit
def kernel(x, adj_t, W, b):
    row = adj_t[0].astype(jnp.int32)
    col = adj_t[1].astype(jnp.int32)
    pad = jnp.full((EP - E,), NP - 1, jnp.int32)
    row_p = jnp.concatenate([row, pad]).reshape(TCHUNKS, CHUNK)
    col_p = jnp.concatenate([col, pad]).reshape(TCHUNKS, CHUNK)
    x_p = jnp.pad(x, ((0, NP - N), (0, 0)))

    degp = _deg_kernel(col_p)
    u0, dinv = _proj(x_p, W, degp)
    p1 = _hop_kernel(u0, row_p, col_p)
    t2 = _mid(p1, u0, dinv)
    p2 = _hop_kernel(t2, row_p, col_p)
    out = _fin(p2, t2, dinv, b)
    return out[:N]


# speedup vs baseline: 23.2221x; 1.0002x over previous
import functools
import jax
import jax.numpy as jnp
from jax import lax
from jax.experimental import pallas as pl
from jax.experimental.pallas import tpu as pltpu
from jax.experimental.pallas import tpu_sc as plsc

N = 10000
E = 320000
D = 128
C = 40

NP = 10240
NC, NS = 2, 16
NW = NC * NS
CHUNK = 128
TCHUNKS = 2560
EP = TCHUNKS * CHUNK
NC0 = 120
NC1 = 40
ROWS_PER_TILE = NP // NS
NBUF = 4


def _mesh():
    return plsc.VectorSubcoreMesh(core_axis_name="c", subcore_axis_name="s")


@functools.partial(
    pl.kernel,
    out_type=jax.ShapeDtypeStruct((NC, NP), jnp.float32),
    mesh=_mesh(),
    scratch_types=[
        pltpu.VMEM((TCHUNKS // NW, CHUNK), jnp.int32),
        pltpu.VMEM((CHUNK,), jnp.float32),
        pltpu.VMEM((ROWS_PER_TILE,), jnp.float32),
        pltpu.VMEM_SHARED((NP,), jnp.float32),
    ],
    compiler_params=pltpu.CompilerParams(use_tc_tiling_on_sc=False),
)
def _deg_kernel(cidx_hbm, out_hbm, cidx_v, ones_v, vbuf, acc_sh):
    cid = lax.axis_index("c")
    sid = lax.axis_index("s")
    wid = sid * NC + cid
    base = sid * ROWS_PER_TILE
    nck = TCHUNKS // NW

    def fill(i, _):
        ones_v[pl.ds(i * 16, 16)] = jnp.ones((16,), jnp.float32)
        return 0
    lax.fori_loop(0, CHUNK // 16, fill, 0)

    def zero(i, _):
        vbuf[pl.ds(i * 16, 16)] = jnp.zeros((16,), jnp.float32)
        return 0
    lax.fori_loop(0, ROWS_PER_TILE // 16, zero, 0)

    pltpu.sync_copy(vbuf, acc_sh.at[pl.ds(base, ROWS_PER_TILE)])
    pltpu.sync_copy(cidx_hbm.at[pl.ds(wid * nck, nck)], cidx_v)
    plsc.subcore_barrier()

    def body(i, _):
        pltpu.sync_copy(ones_v, acc_sh.at[cidx_v.at[i]], add=True)
        return 0
    lax.fori_loop(0, nck, body, 0)

    plsc.subcore_barrier()
    pltpu.sync_copy(acc_sh.at[pl.ds(base, ROWS_PER_TILE)], vbuf)
    pltpu.sync_copy(vbuf, out_hbm.at[cid, pl.ds(base, ROWS_PER_TILE)])


@functools.partial(
    pl.kernel,
    out_type=jax.ShapeDtypeStruct((NC, NP, C), jnp.float32),
    mesh=_mesh(),
    scratch_types=[
        pltpu.VMEM((NC0, CHUNK), jnp.int32),
        pltpu.VMEM((NC0, CHUNK), jnp.int32),
        [pltpu.VMEM((CHUNK, C), jnp.float32) for _ in range(NBUF)],
        [pltpu.SemaphoreType.DMA for _ in range(NBUF)],
        [pltpu.SemaphoreType.DMA for _ in range(NBUF)],
        pltpu.VMEM((ROWS_PER_TILE, C), jnp.float32),
        pltpu.VMEM_SHARED((NP, C), jnp.float32),
    ],
    compiler_params=pltpu.CompilerParams(use_tc_tiling_on_sc=False),
)
def _hop_kernel(u_hbm, ridx_hbm, cidx_hbm, out_hbm, ridx_v, cidx_v, bufs,
                gsems, ssems, vbuf, acc_sh):
    cid = lax.axis_index("c")
    sid = lax.axis_index("s")
    base = sid * ROWS_PER_TILE

    def pipeline(nchunks, off):
        with jax.named_scope("idxcopy"):
            pltpu.sync_copy(ridx_hbm.at[pl.ds(off, nchunks)],
                            ridx_v.at[pl.ds(0, nchunks)])
            pltpu.sync_copy(cidx_hbm.at[pl.ds(off, nchunks)],
                            cidx_v.at[pl.ds(0, nchunks)])
        for b in range(NBUF):
            pltpu.async_copy(u_hbm.at[ridx_v.at[b]], bufs[b], gsems[b])

        with jax.named_scope("accinit"):
            pltpu.sync_copy(u_hbm.at[pl.ds(base, ROWS_PER_TILE)], vbuf)
            pltpu.sync_copy(vbuf, acc_sh.at[pl.ds(base, ROWS_PER_TILE)])
            plsc.subcore_barrier()

        def body(i, _):
            for b in range(NBUF):
                c = i * NBUF + b
                pltpu.make_async_copy(u_hbm.at[ridx_v.at[c]], bufs[b],
                                      gsems[b]).wait()
                pltpu.async_copy(bufs[b], acc_sh.at[cidx_v.at[c]], ssems[b],
                                 add=True)
            for b in range(NBUF):
                c = i * NBUF + b
                cn = jnp.minimum(c + NBUF, nchunks - 1)
                pltpu.make_async_copy(bufs[b], acc_sh.at[cidx_v.at[c]],
                                      ssems[b]).wait()
                pltpu.async_copy(u_hbm.at[ridx_v.at[cn]], bufs[b], gsems[b])
            return 0
        with jax.named_scope("edgeloop"):
            lax.fori_loop(0, nchunks // NBUF, body, 0)

            for b in range(NBUF):
                pltpu.make_async_copy(u_hbm.at[ridx_v.at[nchunks - 1]],
                                      bufs[b], gsems[b]).wait()
        with jax.named_scope("endbar"):
            plsc.subcore_barrier()
        with jax.named_scope("writeout"):
            pltpu.sync_copy(acc_sh.at[pl.ds(base, ROWS_PER_TILE)], vbuf)
            pltpu.sync_copy(vbuf, out_hbm.at[cid, pl.ds(base, ROWS_PER_TILE)])

    @pl.when(cid == 0)
    def _():
        pipeline(NC0, sid * NC0)

    @pl.when(cid == 1)
    def _():
        pipeline(NC1, NS * NC0 + sid * NC1)


def _proj_body(x_ref, w_ref, degp_ref, u0_ref, dinv_ref):
    deg = degp_ref[0, :] + degp_ref[1, :] + 1.0
    dv = lax.rsqrt(deg)
    z = lax.dot_general(x_ref[...], w_ref[...], (((1,), (1,)), ((), ())),
                        preferred_element_type=jnp.float32)
    u0_ref[...] = z * dv[:, None]
    dinv_ref[...] = dv


def _proj(x_p, W, degp):
    blk = 1024
    return pl.pallas_call(
        _proj_body,
        grid=(NP // blk,),
        in_specs=[
            pl.BlockSpec((blk, D), lambda i: (i, 0)),
            pl.BlockSpec((C, D), lambda i: (0, 0)),
            pl.BlockSpec((NC, blk), lambda i: (0, i)),
        ],
        out_specs=[
            pl.BlockSpec((blk, C), lambda i: (i, 0)),
            pl.BlockSpec((blk,), lambda i: (i,)),
        ],
        out_shape=[
            jax.ShapeDtypeStruct((NP, C), jnp.float32),
            jax.ShapeDtypeStruct((NP,), jnp.float32),
        ],
    )(x_p, W, degp)


def _mid_body(p_ref, u_ref, dinv_ref, out_ref):
    s = p_ref[0] + p_ref[1] - u_ref[...]
    dv = dinv_ref[...]
    out_ref[...] = s * (dv * dv)[:, None]


def _mid(parts, u, dinv):
    blk = 1024
    return pl.pallas_call(
        _mid_body,
        grid=(NP // blk,),
        in_specs=[
            pl.BlockSpec((NC, blk, C), lambda i: (0, i, 0)),
            pl.BlockSpec((blk, C), lambda i: (i, 0)),
            pl.BlockSpec((blk,), lambda i: (i,)),
        ],
        out_specs=pl.BlockSpec((blk, C), lambda i: (i, 0)),
        out_shape=jax.ShapeDtypeStruct((NP, C), jnp.float32),
    )(parts, u, dinv)


def _fin_body(p_ref, u_ref, dinv_ref, b_ref, out_ref):
    s = p_ref[0] + p_ref[1] - u_ref[...]
    out_ref[...] = s * dinv_ref[...][:, None] + b_ref[...][None, :]


def _fin(parts, u, dinv, b):
    blk = 1024
    return pl.pallas_call(
        _fin_body,
        grid=(NP // blk,),
        in_specs=[
            pl.BlockSpec((NC, blk, C), lambda i: (0, i, 0)),
            pl.BlockSpec((blk, C), lambda i: (i, 0)),
            pl.BlockSpec((blk,), lambda i: (i,)),
            pl.BlockSpec((C,), lambda i: (0,)),
        ],
        out_specs=pl.BlockSpec((blk, C), lambda i: (i, 0)),
        out_shape=jax.ShapeDtypeStruct((NP, C), jnp.float32),
    )(parts, u, dinv, b)


@jax.jit
def kernel(x, adj_t, W, b):
    row = adj_t[0].astype(jnp.int32)
    col = adj_t[1].astype(jnp.int32)
    pad = jnp.full((EP - E,), NP - 1, jnp.int32)
    row_p = jnp.concatenate([row, pad]).reshape(TCHUNKS, CHUNK)
    col_p = jnp.concatenate([col, pad]).reshape(TCHUNKS, CHUNK)
    x_p = jnp.pad(x, ((0, NP - N), (0, 0)))

    degp = _deg_kernel(col_p)
    u0, dinv = _proj(x_p, W, degp)
    p1 = _hop_kernel(u0, row_p, col_p)
    t2 = _mid(p1, u0, dinv)
    p2 = _hop_kernel(t2, row_p, col_p)
    out = _fin(p2, t2, dinv, b)
    return out[:N]


# speedup vs baseline: 48.1851x; 2.0750x over previous
import functools
import jax
import jax.numpy as jnp
from jax import lax
from jax.experimental import pallas as pl
from jax.experimental.pallas import tpu as pltpu
from jax.experimental.pallas import tpu_sc as plsc

N = 10000
E = 320000
D = 128
C = 40

NP = 10240
NC, NS = 2, 16
NW = NC * NS
CHUNK = 128
TCHUNKS = 2560
EP = TCHUNKS * CHUNK
NC0 = 80
NC1 = 80
ROWS_PER_TILE = NP // NS
NBUF = 4


def _mesh():
    return plsc.VectorSubcoreMesh(core_axis_name="c", subcore_axis_name="s")


@functools.partial(
    pl.kernel,
    out_type=jax.ShapeDtypeStruct((NC, NP), jnp.float32),
    mesh=_mesh(),
    scratch_types=[
        pltpu.VMEM((TCHUNKS // NW, CHUNK), jnp.int32),
        pltpu.VMEM((CHUNK,), jnp.float32),
        pltpu.VMEM((ROWS_PER_TILE,), jnp.float32),
        pltpu.VMEM_SHARED((NP,), jnp.float32),
    ],
    compiler_params=pltpu.CompilerParams(use_tc_tiling_on_sc=False),
)
def _deg_kernel(cidx_hbm, out_hbm, cidx_v, ones_v, vbuf, acc_sh):
    cid = lax.axis_index("c")
    sid = lax.axis_index("s")
    wid = sid * NC + cid
    base = sid * ROWS_PER_TILE
    nck = TCHUNKS // NW

    def fill(i, _):
        ones_v[pl.ds(i * 16, 16)] = jnp.ones((16,), jnp.float32)
        return 0
    lax.fori_loop(0, CHUNK // 16, fill, 0)

    def zero(i, _):
        vbuf[pl.ds(i * 16, 16)] = jnp.zeros((16,), jnp.float32)
        return 0
    lax.fori_loop(0, ROWS_PER_TILE // 16, zero, 0)

    pltpu.sync_copy(vbuf, acc_sh.at[pl.ds(base, ROWS_PER_TILE)])
    pltpu.sync_copy(cidx_hbm.at[pl.ds(wid * nck, nck)], cidx_v)
    plsc.subcore_barrier()

    def body(i, _):
        pltpu.sync_copy(ones_v, acc_sh.at[cidx_v.at[i]], add=True)
        return 0
    lax.fori_loop(0, nck, body, 0)

    plsc.subcore_barrier()
    pltpu.sync_copy(acc_sh.at[pl.ds(base, ROWS_PER_TILE)], vbuf)
    pltpu.sync_copy(vbuf, out_hbm.at[cid, pl.ds(base, ROWS_PER_TILE)])


@functools.partial(
    pl.kernel,
    out_type=jax.ShapeDtypeStruct((NC, NP, C), jnp.float32),
    mesh=_mesh(),
    scratch_types=[
        pltpu.VMEM((NC0, CHUNK), jnp.int32),
        pltpu.VMEM((NC0, CHUNK), jnp.int32),
        [pltpu.VMEM((CHUNK, C), jnp.float32) for _ in range(NBUF)],
        [pltpu.SemaphoreType.DMA for _ in range(NBUF)],
        [pltpu.SemaphoreType.DMA for _ in range(NBUF)],
        pltpu.VMEM((ROWS_PER_TILE, C), jnp.float32),
        pltpu.VMEM_SHARED((NP, C), jnp.float32),
    ],
    compiler_params=pltpu.CompilerParams(use_tc_tiling_on_sc=False),
)
def _hop_kernel(u_hbm, ridx_hbm, cidx_hbm, out_hbm, ridx_v, cidx_v, bufs,
                gsems, ssems, vbuf, acc_sh):
    cid = lax.axis_index("c")
    sid = lax.axis_index("s")
    base = sid * ROWS_PER_TILE

    def pipeline(nchunks, off):
        with jax.named_scope("idxcopy"):
            pltpu.sync_copy(ridx_hbm.at[pl.ds(off, nchunks)],
                            ridx_v.at[pl.ds(0, nchunks)])
            pltpu.sync_copy(cidx_hbm.at[pl.ds(off, nchunks)],
                            cidx_v.at[pl.ds(0, nchunks)])
        for b in range(NBUF):
            pltpu.async_copy(u_hbm.at[ridx_v.at[b]], bufs[b], gsems[b])

        with jax.named_scope("accinit"):
            pltpu.sync_copy(u_hbm.at[pl.ds(base, ROWS_PER_TILE)], vbuf)
            pltpu.sync_copy(vbuf, acc_sh.at[pl.ds(base, ROWS_PER_TILE)])
            plsc.subcore_barrier()

        def body(i, _):
            for b in range(NBUF):
                c = i * NBUF + b
                pltpu.make_async_copy(u_hbm.at[ridx_v.at[c]], bufs[b],
                                      gsems[b]).wait()
                pltpu.async_copy(bufs[b], acc_sh.at[cidx_v.at[c]], ssems[b],
                                 add=True)
            for b in range(NBUF):
                c = i * NBUF + b
                cn = jnp.minimum(c + NBUF, nchunks - 1)
                pltpu.make_async_copy(bufs[b], acc_sh.at[cidx_v.at[c]],
                                      ssems[b]).wait()
                pltpu.async_copy(u_hbm.at[ridx_v.at[cn]], bufs[b], gsems[b])
            return 0
        with jax.named_scope("edgeloop"):
            lax.fori_loop(0, nchunks // NBUF, body, 0)

            for b in range(NBUF):
                pltpu.make_async_copy(u_hbm.at[ridx_v.at[nchunks - 1]],
                                      bufs[b], gsems[b]).wait()
        with jax.named_scope("endbar"):
            plsc.subcore_barrier()
        with jax.named_scope("writeout"):
            pltpu.sync_copy(acc_sh.at[pl.ds(base, ROWS_PER_TILE)], vbuf)
            pltpu.sync_copy(vbuf, out_hbm.at[cid, pl.ds(base, ROWS_PER_TILE)])

    @pl.when(cid == 0)
    def _():
        pipeline(NC0, sid * NC0)

    @pl.when(cid == 1)
    def _():
        pipeline(NC1, NS * NC0 + sid * NC1)


def _proj_body(x_ref, w_ref, degp_ref, u0_ref, dinv_ref):
    deg = degp_ref[0, :] + degp_ref[1, :] + 1.0
    dv = lax.rsqrt(deg)
    z = lax.dot_general(x_ref[...], w_ref[...], (((1,), (1,)), ((), ())),
                        preferred_element_type=jnp.float32)
    u0_ref[...] = z * dv[:, None]
    dinv_ref[...] = dv


def _proj(x_p, W, degp):
    blk = 1024
    return pl.pallas_call(
        _proj_body,
        grid=(NP // blk,),
        in_specs=[
            pl.BlockSpec((blk, D), lambda i: (i, 0)),
            pl.BlockSpec((C, D), lambda i: (0, 0)),
            pl.BlockSpec((NC, blk), lambda i: (0, i)),
        ],
        out_specs=[
            pl.BlockSpec((blk, C), lambda i: (i, 0)),
            pl.BlockSpec((blk,), lambda i: (i,)),
        ],
        out_shape=[
            jax.ShapeDtypeStruct((NP, C), jnp.float32),
            jax.ShapeDtypeStruct((NP,), jnp.float32),
        ],
    )(x_p, W, degp)


def _mid_body(p_ref, u_ref, dinv_ref, out_ref):
    s = p_ref[0] + p_ref[1] - u_ref[...]
    dv = dinv_ref[...]
    out_ref[...] = s * (dv * dv)[:, None]


def _mid(parts, u, dinv):
    blk = 1024
    return pl.pallas_call(
        _mid_body,
        grid=(NP // blk,),
        in_specs=[
            pl.BlockSpec((NC, blk, C), lambda i: (0, i, 0)),
            pl.BlockSpec((blk, C), lambda i: (i, 0)),
            pl.BlockSpec((blk,), lambda i: (i,)),
        ],
        out_specs=pl.BlockSpec((blk, C), lambda i: (i, 0)),
        out_shape=jax.ShapeDtypeStruct((NP, C), jnp.float32),
    )(parts, u, dinv)


def _fin_body(p_ref, u_ref, dinv_ref, b_ref, out_ref):
    s = p_ref[0] + p_ref[1] - u_ref[...]
    out_ref[...] = s * dinv_ref[...][:, None] + b_ref[...][None, :]


def _fin(parts, u, dinv, b):
    blk = 1024
    return pl.pallas_call(
        _fin_body,
        grid=(NP // blk,),
        in_specs=[
            pl.BlockSpec((NC, blk, C), lambda i: (0, i, 0)),
            pl.BlockSpec((blk, C), lambda i: (i, 0)),
            pl.BlockSpec((blk,), lambda i: (i,)),
            pl.BlockSpec((C,), lambda i: (0,)),
        ],
        out_specs=pl.BlockSpec((blk, C), lambda i: (i, 0)),
        out_shape=jax.ShapeDtypeStruct((NP, C), jnp.float32),
    )(parts, u, dinv, b)


@jax.jit
def kernel(x, adj_t, W, b):
    row = adj_t[0].astype(jnp.int32)
    col = adj_t[1].astype(jnp.int32)
    pad = (N + jnp.arange(EP - E, dtype=jnp.int32) % (NP - N)).astype(jnp.int32)
    row_p = jnp.concatenate([row, pad]).reshape(TCHUNKS, CHUNK)
    col_p = jnp.concatenate([col, pad]).reshape(TCHUNKS, CHUNK)
    x_p = jnp.pad(x, ((0, NP - N), (0, 0)))

    degp = _deg_kernel(col_p)
    u0, dinv = _proj(x_p, W, degp)
    p1 = _hop_kernel(u0, row_p, col_p)
    t2 = _mid(p1, u0, dinv)
    p2 = _hop_kernel(t2, row_p, col_p)
    out = _fin(p2, t2, dinv, b)
    return out[:N]


# speedup vs baseline: 50.6543x; 1.0512x over previous
import functools
import jax
import jax.numpy as jnp
from jax import lax
from jax.experimental import pallas as pl
from jax.experimental.pallas import tpu as pltpu
from jax.experimental.pallas import tpu_sc as plsc

N = 10000
E = 320000
D = 128
C = 40

NP = 10240
NC, NS = 2, 16
NW = NC * NS
CHUNK = 128
TCHUNKS = 2560
EP = TCHUNKS * CHUNK
NCK = TCHUNKS // NW
ROWS_PER_TILE = NP // NS
ROWS_PER_W = NP // NW
NBUF = 4


def _mesh():
    return plsc.VectorSubcoreMesh(core_axis_name="c", subcore_axis_name="s")


@functools.partial(
    pl.kernel,
    out_type=jax.ShapeDtypeStruct((NC, NP), jnp.float32),
    mesh=_mesh(),
    scratch_types=[
        pltpu.VMEM((NCK, CHUNK), jnp.int32),
        pltpu.VMEM((CHUNK,), jnp.float32),
        pltpu.VMEM((ROWS_PER_TILE,), jnp.float32),
        pltpu.VMEM_SHARED((NP,), jnp.float32),
    ],
    compiler_params=pltpu.CompilerParams(use_tc_tiling_on_sc=False),
)
def _deg_kernel(cidx_hbm, out_hbm, cidx_v, ones_v, vbuf, acc_sh):
    cid = lax.axis_index("c")
    sid = lax.axis_index("s")
    wid = sid * NC + cid
    base = sid * ROWS_PER_TILE

    def fill(i, _):
        ones_v[pl.ds(i * 16, 16)] = jnp.ones((16,), jnp.float32)
        return 0
    lax.fori_loop(0, CHUNK // 16, fill, 0)

    def zero(i, _):
        vbuf[pl.ds(i * 16, 16)] = jnp.zeros((16,), jnp.float32)
        return 0
    lax.fori_loop(0, ROWS_PER_TILE // 16, zero, 0)

    pltpu.sync_copy(vbuf, acc_sh.at[pl.ds(base, ROWS_PER_TILE)])
    pltpu.sync_copy(cidx_hbm.at[pl.ds(wid * NCK, NCK)], cidx_v)
    plsc.subcore_barrier()

    def body(i, _):
        pltpu.sync_copy(ones_v, acc_sh.at[cidx_v.at[i]], add=True)
        return 0
    lax.fori_loop(0, NCK, body, 0)

    plsc.subcore_barrier()
    pltpu.sync_copy(acc_sh.at[pl.ds(base, ROWS_PER_TILE)], vbuf)
    pltpu.sync_copy(vbuf, out_hbm.at[cid, pl.ds(base, ROWS_PER_TILE)])


@functools.partial(
    pl.kernel,
    out_type=jax.ShapeDtypeStruct((NC, NP, C), jnp.float32),
    mesh=_mesh(),
    scratch_types=[
        pltpu.VMEM((NCK, CHUNK), jnp.int32),
        pltpu.VMEM((NCK, CHUNK), jnp.int32),
        [pltpu.VMEM((CHUNK, C), jnp.float32) for _ in range(NBUF)],
        [pltpu.SemaphoreType.DMA for _ in range(NBUF)],
        [pltpu.SemaphoreType.DMA for _ in range(NBUF)],
        pltpu.VMEM((ROWS_PER_TILE, C), jnp.float32),
        pltpu.VMEM_SHARED((NP, C), jnp.float32),
    ],
    compiler_params=pltpu.CompilerParams(use_tc_tiling_on_sc=False),
)
def _hop_kernel(u_hbm, ridx_hbm, cidx_hbm, out_hbm, ridx_v, cidx_v, bufs,
                gsems, ssems, vbuf, acc_sh):
    cid = lax.axis_index("c")
    sid = lax.axis_index("s")
    wid = sid * NC + cid
    base = sid * ROWS_PER_TILE
    off = wid * NCK

    pltpu.sync_copy(ridx_hbm.at[pl.ds(off, NCK)], ridx_v)
    pltpu.sync_copy(cidx_hbm.at[pl.ds(off, NCK)], cidx_v)
    for b in range(NBUF):
        pltpu.async_copy(u_hbm.at[ridx_v.at[b]], bufs[b], gsems[b])

    pltpu.sync_copy(u_hbm.at[pl.ds(base, ROWS_PER_TILE)], vbuf)
    pltpu.sync_copy(vbuf, acc_sh.at[pl.ds(base, ROWS_PER_TILE)])
    plsc.subcore_barrier()

    def body(i, _):
        for b in range(NBUF):
            c = i * NBUF + b
            pltpu.make_async_copy(u_hbm.at[ridx_v.at[c]], bufs[b],
                                  gsems[b]).wait()
            pltpu.async_copy(bufs[b], acc_sh.at[cidx_v.at[c]], ssems[b],
                             add=True)
        for b in range(NBUF):
            c = i * NBUF + b
            cn = jnp.minimum(c + NBUF, NCK - 1)
            pltpu.make_async_copy(bufs[b], acc_sh.at[cidx_v.at[c]],
                                  ssems[b]).wait()
            pltpu.async_copy(u_hbm.at[ridx_v.at[cn]], bufs[b], gsems[b])
        return 0
    lax.fori_loop(0, NCK // NBUF, body, 0)

    for b in range(NBUF):
        pltpu.make_async_copy(u_hbm.at[ridx_v.at[NCK - 1]], bufs[b],
                              gsems[b]).wait()
    plsc.subcore_barrier()
    pltpu.sync_copy(acc_sh.at[pl.ds(base, ROWS_PER_TILE)], vbuf)
    pltpu.sync_copy(vbuf, out_hbm.at[cid, pl.ds(base, ROWS_PER_TILE)])


def _combine_body(square, with_bias):
    def body(p_hbm, u_hbm, dinv_hbm, *rest):
        if with_bias:
            b_hbm, out_hbm, pv0, pv1, uv, dv, bv, ov = rest
        else:
            out_hbm, pv0, pv1, uv, dv, ov = rest
        cid = lax.axis_index("c")
        sid = lax.axis_index("s")
        wid = sid * NC + cid
        r0 = wid * ROWS_PER_W
        pltpu.sync_copy(p_hbm.at[0, pl.ds(r0, ROWS_PER_W)], pv0)
        pltpu.sync_copy(p_hbm.at[1, pl.ds(r0, ROWS_PER_W)], pv1)
        pltpu.sync_copy(u_hbm.at[pl.ds(r0, ROWS_PER_W)], uv)
        pltpu.sync_copy(dinv_hbm.at[pl.ds(r0, ROWS_PER_W)], dv)
        if with_bias:
            pltpu.sync_copy(b_hbm, bv)
            bvecs = {o: bv[pl.ds(o, 16)] for o in (0, 16, 24)}
        else:
            bvecs = {o: None for o in (0, 16, 24)}

        def rowfn(r, _):
            s = plsc.load_gather(dv, [jnp.full((16,), r, jnp.int32)])
            if square:
                s = s * s
            for o in (0, 16, 24):
                v = (pv0[r, pl.ds(o, 16)] + pv1[r, pl.ds(o, 16)]
                     - uv[r, pl.ds(o, 16)]) * s
                if with_bias:
                    v = v + bvecs[o]
                ov[r, pl.ds(o, 16)] = v
            return 0
        lax.fori_loop(0, ROWS_PER_W, rowfn, 0)
        pltpu.sync_copy(ov, out_hbm.at[pl.ds(r0, ROWS_PER_W)])
    return body


_combine_scratch = [
    pltpu.VMEM((ROWS_PER_W, C), jnp.float32),
    pltpu.VMEM((ROWS_PER_W, C), jnp.float32),
    pltpu.VMEM((ROWS_PER_W, C), jnp.float32),
    pltpu.VMEM((ROWS_PER_W,), jnp.float32),
]

_mid_kernel = functools.partial(
    pl.kernel,
    out_type=jax.ShapeDtypeStruct((NP, C), jnp.float32),
    mesh=_mesh(),
    scratch_types=_combine_scratch + [
        pltpu.VMEM((ROWS_PER_W, C), jnp.float32),
    ],
    compiler_params=pltpu.CompilerParams(use_tc_tiling_on_sc=False,
                                         needs_layout_passes=False),
)(_combine_body(square=True, with_bias=False))

_fin_kernel = functools.partial(
    pl.kernel,
    out_type=jax.ShapeDtypeStruct((NP, C), jnp.float32),
    mesh=_mesh(),
    scratch_types=_combine_scratch + [
        pltpu.VMEM((C,), jnp.float32),
        pltpu.VMEM((ROWS_PER_W, C), jnp.float32),
    ],
    compiler_params=pltpu.CompilerParams(use_tc_tiling_on_sc=False,
                                         needs_layout_passes=False),
)(_combine_body(square=False, with_bias=True))


def _matmul_body(x_ref, w_ref, z_ref):
    z_ref[...] = lax.dot_general(x_ref[...], w_ref[...],
                                 (((1,), (1,)), ((), ())),
                                 preferred_element_type=jnp.float32)


def _matmul(x_p, W):
    blk = 1024
    return pl.pallas_call(
        _matmul_body,
        grid=(NP // blk,),
        in_specs=[
            pl.BlockSpec((blk, D), lambda i: (i, 0)),
            pl.BlockSpec((C, D), lambda i: (0, 0)),
        ],
        out_specs=pl.BlockSpec((blk, C), lambda i: (i, 0)),
        out_shape=jax.ShapeDtypeStruct((NP, C), jnp.float32),
    )(x_p, W)


def _scale_body(z_ref, degp_ref, u0_ref, dinv_ref):
    deg = degp_ref[0, :] + degp_ref[1, :] + 1.0
    dv = lax.rsqrt(deg)
    u0_ref[...] = z_ref[...] * dv[:, None]
    dinv_ref[...] = dv


def _scale(z, degp):
    blk = 1024
    return pl.pallas_call(
        _scale_body,
        grid=(NP // blk,),
        in_specs=[
            pl.BlockSpec((blk, C), lambda i: (i, 0)),
            pl.BlockSpec((NC, blk), lambda i: (0, i)),
        ],
        out_specs=[
            pl.BlockSpec((blk, C), lambda i: (i, 0)),
            pl.BlockSpec((blk,), lambda i: (i,)),
        ],
        out_shape=[
            jax.ShapeDtypeStruct((NP, C), jnp.float32),
            jax.ShapeDtypeStruct((NP,), jnp.float32),
        ],
    )(z, degp)


@jax.jit
def kernel(x, adj_t, W, b):
    row = adj_t[0].astype(jnp.int32)
    col = adj_t[1].astype(jnp.int32)
    pad = (N + jnp.arange(EP - E, dtype=jnp.int32) % (NP - N)).astype(jnp.int32)
    row_p = jnp.concatenate([row, pad]).reshape(TCHUNKS, CHUNK)
    col_p = jnp.concatenate([col, pad]).reshape(TCHUNKS, CHUNK)
    x_p = jnp.pad(x, ((0, NP - N), (0, 0)))

    degp = _deg_kernel(col_p)
    z = _matmul(x_p, W)
    u0, dinv = _scale(z, degp)
    p1 = _hop_kernel(u0, row_p, col_p)
    t2 = _mid_kernel(p1, u0, dinv)
    p2 = _hop_kernel(t2, row_p, col_p)
    out = _fin_kernel(p2, t2, dinv, b)
    return out[:N]
